# Initial kernel scaffold; baseline (speedup 1.0000x reference)
import jax
import jax.numpy as jnp
from jax.experimental import pallas as pl


def kernel(review_feats, user_feats, item_feats, user_idx, item_idx, W):
    raise NotImplementedError("write your pallas kernel here")


# speedup vs baseline: 1.6663x; 1.6663x over previous
import functools

import numpy as np
import jax
import jax.numpy as jnp
from jax import lax
from jax.experimental import pallas as pl
from jax.experimental.pallas import tpu as pltpu
from jax.experimental.pallas import tpu_sc as plsc

D = 128
NV = 50000
NB = 100000
B_PAD = 102400
NC, NS = 2, 16
NW = NC * NS
PER_W = B_PAD // NW
CHUNK = 128
N_CHUNKS = PER_W // CHUNK


_INV_PU = np.array([
    36, 58, 29, 7, 81, 105, 42, 113, 57, 115, 18, 3, 125, 93, 78, 102, 22,
    27, 10, 76, 60, 24, 95, 31, 87, 96, 127, 116, 92, 111, 101, 47, 40, 32,
    69, 28, 61, 122, 85, 37, 118, 51, 44, 34, 21, 79, 80, 73, 26, 119, 56,
    110, 52, 54, 1, 124, 67, 11, 41, 63, 12, 15, 23, 114, 121, 112, 45, 50,
    74, 108, 9, 75, 20, 48, 82, 86, 35, 38, 65, 59, 49, 55, 103, 97, 71, 33,
    5, 46, 4, 83, 106, 72, 70, 8, 0, 2, 98, 100, 84, 99, 25, 64, 94, 53,
    123, 13, 107, 43, 90, 6, 66, 89, 88, 17, 39, 77, 68, 104, 91, 126, 117,
    109, 14, 120, 19, 62, 16, 30], dtype=np.int32)
_INV_PI = np.array([
    102, 18, 20, 39, 35, 104, 13, 38, 87, 98, 82, 125, 103, 59, 33, 100,
    123, 26, 70, 42, 69, 99, 68, 90, 46, 56, 111, 63, 15, 121, 14, 126, 28,
    16, 110, 4, 113, 22, 65, 106, 57, 72, 54, 41, 62, 24, 48, 52, 29, 91,
    74, 107, 58, 21, 76, 124, 31, 12, 19, 0, 67, 79, 95, 17, 50, 45, 10, 96,
    120, 34, 23, 47, 73, 44, 92, 115, 32, 2, 75, 81, 117, 66, 97, 101, 105,
    53, 127, 83, 118, 108, 114, 71, 89, 36, 86, 1, 27, 11, 88, 77, 112, 85,
    5, 84, 49, 43, 60, 9, 37, 64, 8, 3, 109, 122, 55, 119, 61, 51, 30, 7,
    40, 80, 78, 116, 94, 25, 6, 93], dtype=np.int32)


def _proj_body(u_ref, i_ref, w2_ref, w3_ref, up_ref, ip_ref):
    up_ref[...] = jnp.dot(u_ref[...], w2_ref[...],
                          preferred_element_type=jnp.float32)
    ip_ref[...] = jnp.dot(i_ref[...], w3_ref[...],
                          preferred_element_type=jnp.float32)


def _project_tables(user_feats, item_feats, w2p, w3p):
    blk = 2000
    return pl.pallas_call(
        _proj_body,
        grid=(NV // blk,),
        in_specs=[
            pl.BlockSpec((blk, D), lambda i: (i, 0)),
            pl.BlockSpec((blk, D), lambda i: (i, 0)),
            pl.BlockSpec((D, D), lambda i: (0, 0)),
            pl.BlockSpec((D, D), lambda i: (0, 0)),
        ],
        out_specs=[pl.BlockSpec((blk, D), lambda i: (i, 0)),
                   pl.BlockSpec((blk, D), lambda i: (i, 0))],
        out_shape=[jax.ShapeDtypeStruct((NV, D), jnp.float32),
                   jax.ShapeDtypeStruct((NV, D), jnp.float32)],
    )(user_feats, item_feats, w2p, w3p)


def _sc_gather2(up, ipj, ui, ii):
    mesh = plsc.VectorSubcoreMesh(core_axis_name="c", subcore_axis_name="s")

    @functools.partial(
        pl.kernel,
        out_type=(jax.ShapeDtypeStruct((B_PAD, D), jnp.float32),
                  jax.ShapeDtypeStruct((B_PAD, D), jnp.float32)),
        mesh=mesh,
        scratch_types=[
            pltpu.VMEM((CHUNK,), jnp.int32),
            pltpu.VMEM((CHUNK,), jnp.int32),
            pltpu.VMEM((CHUNK, D), jnp.float32),
            pltpu.VMEM((CHUNK, D), jnp.float32),
            pltpu.SemaphoreType.DMA,
            pltpu.SemaphoreType.DMA,
        ],
    )
    def k(up_hbm, ip_hbm, ui_hbm, ii_hbm, gu_hbm, gi_hbm,
          idxu, idxi, rows_u, rows_i, sem_u, sem_i):
        wid = lax.axis_index("s") * NC + lax.axis_index("c")
        base = wid * PER_W

        def chunk(j, carry):
            off = base + j * CHUNK
            pltpu.sync_copy(ui_hbm.at[pl.ds(off, CHUNK)], idxu)
            pltpu.sync_copy(ii_hbm.at[pl.ds(off, CHUNK)], idxi)
            cu = pltpu.async_copy(up_hbm.at[idxu], rows_u, sem_u)
            ci = pltpu.async_copy(ip_hbm.at[idxi], rows_i, sem_i)
            cu.wait()
            ci.wait()
            pltpu.sync_copy(rows_u, gu_hbm.at[pl.ds(off, CHUNK)])
            pltpu.sync_copy(rows_i, gi_hbm.at[pl.ds(off, CHUNK)])
            return carry

        lax.fori_loop(0, N_CHUNKS, chunk, 0)

    return k(up, ipj, ui, ii)


def _final_body(r_ref, gu_ref, gi_ref, w1_ref, o_ref):
    acc = jnp.dot(r_ref[...], w1_ref[...], preferred_element_type=jnp.float32)
    o_ref[...] = jnp.maximum(acc + gu_ref[...] + gi_ref[...], 0.0)


def _final(review, gu, gi, w1):
    blk = 1000
    return pl.pallas_call(
        _final_body,
        grid=(NB // blk,),
        in_specs=[
            pl.BlockSpec((blk, D), lambda i: (i, 0)),
            pl.BlockSpec((blk, D), lambda i: (i, 0)),
            pl.BlockSpec((blk, D), lambda i: (i, 0)),
            pl.BlockSpec((D, D), lambda i: (0, 0)),
        ],
        out_specs=pl.BlockSpec((blk, D), lambda i: (i, 0)),
        out_shape=jax.ShapeDtypeStruct((NB, D), jnp.float32),
    )(review, gu, gi, w1)


def kernel(review_feats, user_feats, item_feats, user_idx, item_idx, W):
    w1 = W[:D]
    w2p = W[D:2 * D][_INV_PU]
    w3p = W[2 * D:3 * D][_INV_PI]
    ui = jnp.pad(user_idx.astype(jnp.int32), (0, B_PAD - NB))
    ii = jnp.pad(item_idx.astype(jnp.int32), (0, B_PAD - NB))
    up, ipj = _project_tables(user_feats, item_feats, w2p, w3p)
    gu, gi = _sc_gather2(up, ipj, ui, ii)
    return _final(review_feats, gu, gi, w1)


# speedup vs baseline: 1.9366x; 1.1622x over previous
import functools

import numpy as np
import jax
import jax.numpy as jnp
from jax import lax
from jax.experimental import pallas as pl
from jax.experimental.pallas import tpu as pltpu
from jax.experimental.pallas import tpu_sc as plsc

D = 128
NV = 50000
NB = 100000
B_PAD = 102400
NC, NS = 2, 16
NW = NC * NS
PER_W = B_PAD // NW
CHUNK = 128
N_CHUNKS = PER_W // CHUNK


_INV_PU = np.array([
    36, 58, 29, 7, 81, 105, 42, 113, 57, 115, 18, 3, 125, 93, 78, 102, 22,
    27, 10, 76, 60, 24, 95, 31, 87, 96, 127, 116, 92, 111, 101, 47, 40, 32,
    69, 28, 61, 122, 85, 37, 118, 51, 44, 34, 21, 79, 80, 73, 26, 119, 56,
    110, 52, 54, 1, 124, 67, 11, 41, 63, 12, 15, 23, 114, 121, 112, 45, 50,
    74, 108, 9, 75, 20, 48, 82, 86, 35, 38, 65, 59, 49, 55, 103, 97, 71, 33,
    5, 46, 4, 83, 106, 72, 70, 8, 0, 2, 98, 100, 84, 99, 25, 64, 94, 53,
    123, 13, 107, 43, 90, 6, 66, 89, 88, 17, 39, 77, 68, 104, 91, 126, 117,
    109, 14, 120, 19, 62, 16, 30], dtype=np.int32)
_INV_PI = np.array([
    102, 18, 20, 39, 35, 104, 13, 38, 87, 98, 82, 125, 103, 59, 33, 100,
    123, 26, 70, 42, 69, 99, 68, 90, 46, 56, 111, 63, 15, 121, 14, 126, 28,
    16, 110, 4, 113, 22, 65, 106, 57, 72, 54, 41, 62, 24, 48, 52, 29, 91,
    74, 107, 58, 21, 76, 124, 31, 12, 19, 0, 67, 79, 95, 17, 50, 45, 10, 96,
    120, 34, 23, 47, 73, 44, 92, 115, 32, 2, 75, 81, 117, 66, 97, 101, 105,
    53, 127, 83, 118, 108, 114, 71, 89, 36, 86, 1, 27, 11, 88, 77, 112, 85,
    5, 84, 49, 43, 60, 9, 37, 64, 8, 3, 109, 122, 55, 119, 61, 51, 30, 7,
    40, 80, 78, 116, 94, 25, 6, 93], dtype=np.int32)


def _proj_body(u_ref, i_ref, w2_ref, w3_ref, up_ref, ip_ref):
    up_ref[...] = jnp.dot(u_ref[...], w2_ref[...],
                          preferred_element_type=jnp.float32)
    ip_ref[...] = jnp.dot(i_ref[...], w3_ref[...],
                          preferred_element_type=jnp.float32)


def _project_tables(user_feats, item_feats, w2p, w3p):
    blk = 2000
    return pl.pallas_call(
        _proj_body,
        grid=(NV // blk,),
        in_specs=[
            pl.BlockSpec((blk, D), lambda i: (i, 0)),
            pl.BlockSpec((blk, D), lambda i: (i, 0)),
            pl.BlockSpec((D, D), lambda i: (0, 0)),
            pl.BlockSpec((D, D), lambda i: (0, 0)),
        ],
        out_specs=[pl.BlockSpec((blk, D), lambda i: (i, 0)),
                   pl.BlockSpec((blk, D), lambda i: (i, 0))],
        out_shape=[jax.ShapeDtypeStruct((NV, D), jnp.float32),
                   jax.ShapeDtypeStruct((NV, D), jnp.float32)],
    )(user_feats, item_feats, w2p, w3p)


NBUF = 3


def _sc_gather2(up, ipj, ui, ii):
    mesh = plsc.VectorSubcoreMesh(core_axis_name="c", subcore_axis_name="s")

    @functools.partial(
        pl.kernel,
        out_type=(jax.ShapeDtypeStruct((B_PAD, D), jnp.float32),
                  jax.ShapeDtypeStruct((B_PAD, D), jnp.float32)),
        mesh=mesh,
        scratch_types=(
            [pltpu.VMEM((N_CHUNKS, CHUNK), jnp.int32),
             pltpu.VMEM((N_CHUNKS, CHUNK), jnp.int32),
             pltpu.VMEM((NBUF, CHUNK, D), jnp.float32),
             pltpu.VMEM((NBUF, CHUNK, D), jnp.float32)]
            + [pltpu.SemaphoreType.DMA] * (4 * NBUF)
        ),
    )
    def k(up_hbm, ip_hbm, ui_hbm, ii_hbm, gu_hbm, gi_hbm,
          idxu, idxi, rows_u, rows_i, *sems):
        sem_gu = sems[0:NBUF]
        sem_gi = sems[NBUF:2 * NBUF]
        sem_ou = sems[2 * NBUF:3 * NBUF]
        sem_oi = sems[3 * NBUF:4 * NBUF]
        wid = lax.axis_index("s") * NC + lax.axis_index("c")
        base = wid * PER_W
        pltpu.sync_copy(ui_hbm.at[wid], idxu)
        pltpu.sync_copy(ii_hbm.at[wid], idxi)

        gat = [None] * NBUF
        out = [None] * NBUF

        def issue_out(j):
            pb = j % NBUF
            gcu, gci = gat[pb]
            gcu.wait()
            gci.wait()
            off = base + j * CHUNK
            out[pb] = (
                pltpu.async_copy(rows_u.at[pb], gu_hbm.at[pl.ds(off, CHUNK)],
                                 sem_ou[pb]),
                pltpu.async_copy(rows_i.at[pb], gi_hbm.at[pl.ds(off, CHUNK)],
                                 sem_oi[pb]),
            )

        for j in range(N_CHUNKS):
            b = j % NBUF
            if out[b] is not None:
                out[b][0].wait()
                out[b][1].wait()
                out[b] = None
            gat[b] = (
                pltpu.async_copy(up_hbm.at[idxu.at[j]], rows_u.at[b],
                                 sem_gu[b]),
                pltpu.async_copy(ip_hbm.at[idxi.at[j]], rows_i.at[b],
                                 sem_gi[b]),
            )
            if j >= 1:
                issue_out(j - 1)
        issue_out(N_CHUNKS - 1)
        for p in out:
            if p is not None:
                p[0].wait()
                p[1].wait()

    return k(up, ipj, ui, ii)


def _final_body(r_ref, gu_ref, gi_ref, w1_ref, o_ref):
    acc = jnp.dot(r_ref[...], w1_ref[...], preferred_element_type=jnp.float32)
    o_ref[...] = jnp.maximum(acc + gu_ref[...] + gi_ref[...], 0.0)


def _final(review, gu, gi, w1):
    blk = 1000
    return pl.pallas_call(
        _final_body,
        grid=(NB // blk,),
        in_specs=[
            pl.BlockSpec((blk, D), lambda i: (i, 0)),
            pl.BlockSpec((blk, D), lambda i: (i, 0)),
            pl.BlockSpec((blk, D), lambda i: (i, 0)),
            pl.BlockSpec((D, D), lambda i: (0, 0)),
        ],
        out_specs=pl.BlockSpec((blk, D), lambda i: (i, 0)),
        out_shape=jax.ShapeDtypeStruct((NB, D), jnp.float32),
    )(review, gu, gi, w1)


def kernel(review_feats, user_feats, item_feats, user_idx, item_idx, W):
    w1 = W[:D]
    w2p = W[D:2 * D][_INV_PU]
    w3p = W[2 * D:3 * D][_INV_PI]
    ui = jnp.pad(user_idx.astype(jnp.int32),
                 (0, B_PAD - NB)).reshape(NW, N_CHUNKS, CHUNK)
    ii = jnp.pad(item_idx.astype(jnp.int32),
                 (0, B_PAD - NB)).reshape(NW, N_CHUNKS, CHUNK)
    up, ipj = _project_tables(user_feats, item_feats, w2p, w3p)
    gu, gi = _sc_gather2(up, ipj, ui, ii)
    return _final(review_feats, gu, gi, w1)


# speedup vs baseline: 3.3858x; 1.7483x over previous
import functools

import numpy as np
import jax
import jax.numpy as jnp
from jax import lax
from jax.experimental import pallas as pl
from jax.experimental.pallas import tpu as pltpu
from jax.experimental.pallas import tpu_sc as plsc

D = 128
NV = 50000
NB = 100000
B_PAD = 102400
NC, NS = 2, 16
PER_S = B_PAD // NS
CHUNK = 256
N_CHUNKS = PER_S // CHUNK
NBUF = 3

_INV_PU = np.array([
    36, 58, 29, 7, 81, 105, 42, 113, 57, 115, 18, 3, 125, 93, 78, 102, 22,
    27, 10, 76, 60, 24, 95, 31, 87, 96, 127, 116, 92, 111, 101, 47, 40, 32,
    69, 28, 61, 122, 85, 37, 118, 51, 44, 34, 21, 79, 80, 73, 26, 119, 56,
    110, 52, 54, 1, 124, 67, 11, 41, 63, 12, 15, 23, 114, 121, 112, 45, 50,
    74, 108, 9, 75, 20, 48, 82, 86, 35, 38, 65, 59, 49, 55, 103, 97, 71, 33,
    5, 46, 4, 83, 106, 72, 70, 8, 0, 2, 98, 100, 84, 99, 25, 64, 94, 53,
    123, 13, 107, 43, 90, 6, 66, 89, 88, 17, 39, 77, 68, 104, 91, 126, 117,
    109, 14, 120, 19, 62, 16, 30], dtype=np.int32)
_INV_PI = np.array([
    102, 18, 20, 39, 35, 104, 13, 38, 87, 98, 82, 125, 103, 59, 33, 100,
    123, 26, 70, 42, 69, 99, 68, 90, 46, 56, 111, 63, 15, 121, 14, 126, 28,
    16, 110, 4, 113, 22, 65, 106, 57, 72, 54, 41, 62, 24, 48, 52, 29, 91,
    74, 107, 58, 21, 76, 124, 31, 12, 19, 0, 67, 79, 95, 17, 50, 45, 10, 96,
    120, 34, 23, 47, 73, 44, 92, 115, 32, 2, 75, 81, 117, 66, 97, 101, 105,
    53, 127, 83, 118, 108, 114, 71, 89, 36, 86, 1, 27, 11, 88, 77, 112, 85,
    5, 84, 49, 43, 60, 9, 37, 64, 8, 3, 109, 122, 55, 119, 61, 51, 30, 7,
    40, 80, 78, 116, 94, 25, 6, 93], dtype=np.int32)


def _proj_body(u_ref, i_ref, w2_ref, w3_ref, up_ref, ip_ref):
    up_ref[...] = jnp.dot(u_ref[...], w2_ref[...],
                          preferred_element_type=jnp.float32)
    ip_ref[...] = jnp.dot(i_ref[...], w3_ref[...],
                          preferred_element_type=jnp.float32)


def _project_tables(user_feats, item_feats, w2p, w3p):
    blk = 2000
    return pl.pallas_call(
        _proj_body,
        grid=(NV // blk,),
        in_specs=[
            pl.BlockSpec((blk, D), lambda i: (i, 0)),
            pl.BlockSpec((blk, D), lambda i: (i, 0)),
            pl.BlockSpec((D, D), lambda i: (0, 0)),
            pl.BlockSpec((D, D), lambda i: (0, 0)),
        ],
        out_specs=[pl.BlockSpec((blk, D), lambda i: (i, 0)),
                   pl.BlockSpec((blk, D), lambda i: (i, 0))],
        out_shape=[jax.ShapeDtypeStruct((NV, D), jnp.float32),
                   jax.ShapeDtypeStruct((NV, D), jnp.float32)],
    )(user_feats, item_feats, w2p, w3p)


def _sc_gather2(up, ipj, ui, ii):
    mesh = plsc.VectorSubcoreMesh(core_axis_name="c", subcore_axis_name="s")

    @functools.partial(
        pl.kernel,
        out_type=(jax.ShapeDtypeStruct((B_PAD, D), jnp.float32),
                  jax.ShapeDtypeStruct((B_PAD, D), jnp.float32)),
        mesh=mesh,
        scratch_types=(
            [pltpu.VMEM((PER_S // D, D), jnp.int32),
             pltpu.VMEM((NBUF, CHUNK, D), jnp.float32)]
            + [pltpu.SemaphoreType.DMA] * (2 * NBUF)
        ),
    )
    def k(up_hbm, ip_hbm, ui_hbm, ii_hbm, gu_hbm, gi_hbm, idx, rows, *sems):
        sem_g = sems[0:NBUF]
        sem_o = sems[NBUF:2 * NBUF]
        core = lax.axis_index("c")
        sl = lax.axis_index("s")
        base = sl * PER_S

        def run(tab, idx_hbm, out_hbm):
            pltpu.sync_copy(idx_hbm.at[sl], idx)
            g = [None] * N_CHUNKS
            w = [None] * N_CHUNKS

            def issue_g(j):
                b = j % NBUF
                g[j] = (
                    pltpu.async_copy(tab.at[idx.at[2 * j]],
                                     rows.at[b, pl.ds(0, D)], sem_g[b]),
                    pltpu.async_copy(tab.at[idx.at[2 * j + 1]],
                                     rows.at[b, pl.ds(D, D)], sem_g[b]),
                )

            issue_g(0)
            issue_g(1)
            for j in range(N_CHUNKS):
                b = j % NBUF
                g[j][0].wait()
                g[j][1].wait()
                w[j] = pltpu.async_copy(
                    rows.at[b], out_hbm.at[pl.ds(base + j * CHUNK, CHUNK)],
                    sem_o[b])
                if j + 2 < N_CHUNKS:
                    if j >= 1:
                        w[j - 1].wait()
                    issue_g(j + 2)
            w[N_CHUNKS - 2].wait()
            w[N_CHUNKS - 1].wait()

        @pl.when(core == 0)
        def _():
            run(up_hbm, ui_hbm, gu_hbm)

        @pl.when(core == 1)
        def _():
            run(ip_hbm, ii_hbm, gi_hbm)

    return k(up, ipj, ui, ii)


def _final_body(r_ref, gu_ref, gi_ref, w1_ref, o_ref):
    acc = jnp.dot(r_ref[...], w1_ref[...], preferred_element_type=jnp.float32)
    o_ref[...] = jnp.maximum(acc + gu_ref[...] + gi_ref[...], 0.0)


def _final(review, gu, gi, w1):
    blk = 2000
    return pl.pallas_call(
        _final_body,
        grid=(NB // blk,),
        in_specs=[
            pl.BlockSpec((blk, D), lambda i: (i, 0)),
            pl.BlockSpec((blk, D), lambda i: (i, 0)),
            pl.BlockSpec((blk, D), lambda i: (i, 0)),
            pl.BlockSpec((D, D), lambda i: (0, 0)),
        ],
        out_specs=pl.BlockSpec((blk, D), lambda i: (i, 0)),
        out_shape=jax.ShapeDtypeStruct((NB, D), jnp.float32),
    )(review, gu, gi, w1)


def _pad_idx(idx):
    pad = jnp.arange(B_PAD - NB, dtype=jnp.int32) % NV
    return jnp.concatenate([idx.astype(jnp.int32), pad]).reshape(
        NS, PER_S // D, D)


def kernel(review_feats, user_feats, item_feats, user_idx, item_idx, W):
    w1 = W[:D]
    w2p = W[D:2 * D][_INV_PU]
    w3p = W[2 * D:3 * D][_INV_PI]
    ui = _pad_idx(user_idx)
    ii = _pad_idx(item_idx)
    up, ipj = _project_tables(user_feats, item_feats, w2p, w3p)
    gu, gi = _sc_gather2(up, ipj, ui, ii)
    return _final(review_feats, gu, gi, w1)
